# baseline (speedup 1.0000x reference)
import functools

import jax
import jax.numpy as jnp
from jax import lax
from jax.experimental import pallas as pl
from jax.experimental.pallas import tpu as pltpu
from jax.experimental.pallas import tpu_sc as plsc

HIDDEN = 128
LN_EPS = 1e-12

_info = plsc.get_sparse_core_info()
_NC, _NS, _L = _info.num_cores, _info.num_subcores, _info.num_lanes
_NW = _NC * _NS

_GATHER_DNUMS = lax.GatherDimensionNumbers(
    offset_dims=(), collapsed_slice_dims=(0,), start_index_map=(0,))


def _lane_shuffle(v, idx):
    return lax.gather(v, idx.reshape(16, 1), _GATHER_DNUMS, (1,),
                      mode=lax.GatherScatterMode.PROMISE_IN_BOUNDS)


def _xlane_sum(v):
    for sh in (8, 4, 2, 1):
        idx = lax.iota(jnp.int32, 16) ^ sh
        v = v + _lane_shuffle(v, idx)
    return v


def _ln_row(rv, pv, r, gj, bj):
    x = [rv[r, pl.ds(16 * j, 16)] for j in range(8)]
    s1v = ((x[0] + x[1]) + (x[2] + x[3])) + ((x[4] + x[5]) + (x[6] + x[7]))
    s2v = ((x[0] * x[0] + x[1] * x[1]) + (x[2] * x[2] + x[3] * x[3])) + \
          ((x[4] * x[4] + x[5] * x[5]) + (x[6] * x[6] + x[7] * x[7]))
    s1 = _xlane_sum(s1v)
    s2 = _xlane_sum(s2v)
    mv = s1 * (1.0 / HIDDEN)
    vv = jnp.maximum(s2 * (1.0 / HIDDEN) - mv * mv, 0.0) + LN_EPS
    bi = lax.bitcast_convert_type(vv, jnp.int32)
    bi = jnp.int32(0x5F3759DF) - (bi >> 1)
    y = lax.bitcast_convert_type(bi, jnp.float32)
    hv = 0.5 * vv
    y = y * (1.5 - hv * (y * y))
    y = y * (1.5 - hv * (y * y))
    o = [(x[j] - mv) * (y * gj[j]) + bj[j] for j in range(8)]
    for m in range(4):
        ua = lax.bitcast_convert_type(o[2 * m], jnp.int32)
        ub = lax.bitcast_convert_type(o[2 * m + 1], jnp.int32)
        lo = lax.shift_right_logical(ua + 0x8000, 16)
        hi = (ub + 0x8000) & jnp.int32(-65536)
        pv[r, pl.ds(16 * m, 16)] = lo | hi


_P1_CHUNK = 160
_P1_NBUF = 3
_P1_AHEAD = 1
_P1_UNROLL = 4


def _make_normalize_table(vocab: int):
    assert vocab % _P1_CHUNK == 0 and _P1_CHUNK % 8 == 0
    n_chunks = vocab // _P1_CHUNK
    max_per_w = -(-n_chunks // _NW)
    n_groups = -(-(max_per_w + _P1_NBUF - _P1_AHEAD) // _P1_NBUF)

    mesh = plsc.VectorSubcoreMesh(core_axis_name="c", subcore_axis_name="s")

    @functools.partial(
        pl.kernel,
        mesh=mesh,
        compiler_params=pltpu.CompilerParams(use_tc_tiling_on_sc=False),
        out_type=jax.ShapeDtypeStruct((vocab, HIDDEN // 2), jnp.int32),
        scratch_types=[
            pltpu.VMEM((_P1_CHUNK, HIDDEN), jnp.float32)
            for _ in range(_P1_NBUF)
        ] + [
            pltpu.VMEM((_P1_CHUNK, HIDDEN // 2), jnp.int32)
            for _ in range(_P1_NBUF)
        ] + [
            pltpu.VMEM((HIDDEN,), jnp.float32),
            pltpu.VMEM((HIDDEN,), jnp.float32),
        ] + [pltpu.SemaphoreType.DMA for _ in range(2 * _P1_NBUF)],
    )
    def normalize_table(table_hbm, gamma_hbm, beta_hbm, norm_hbm,
                        *bufs_and_sems):
        bufs = list(bufs_and_sems[:_P1_NBUF])
        pbufs = list(bufs_and_sems[_P1_NBUF:2 * _P1_NBUF])
        gamma_v, beta_v = bufs_and_sems[2 * _P1_NBUF:2 * _P1_NBUF + 2]
        sems = bufs_and_sems[2 * _P1_NBUF + 2:]
        sem_g = sems[:_P1_NBUF]
        sem_w = sems[_P1_NBUF:]
        wid = lax.axis_index("s") * _NC + lax.axis_index("c")
        n_w = (n_chunks - wid + _NW - 1) // _NW
        pltpu.sync_copy(gamma_hbm, gamma_v)
        pltpu.sync_copy(beta_hbm, beta_v)
        gj = [gamma_v[pl.ds(16 * j, 16)] for j in range(8)]
        bj = [beta_v[pl.ds(16 * j, 16)] for j in range(8)]

        def row_off(c):
            return (wid + c * _NW) * _P1_CHUNK

        def start_load(c, b):
            pltpu.make_async_copy(
                table_hbm.at[pl.ds(row_off(c), _P1_CHUNK)],
                bufs[b], sem_g[b]).start()

        def wait_load(c, b):
            pltpu.make_async_copy(
                table_hbm.at[pl.ds(row_off(c), _P1_CHUNK)],
                bufs[b], sem_g[b]).wait()

        def start_wb(c, b):
            pltpu.make_async_copy(
                pbufs[b], norm_hbm.at[pl.ds(row_off(c), _P1_CHUNK)],
                sem_w[b]).start()

        def wait_wb(b):
            pltpu.make_async_copy(
                pbufs[b], norm_hbm.at[pl.ds(wid * _P1_CHUNK, _P1_CHUNK)],
                sem_w[b]).wait()

        for c in range(_P1_AHEAD):

            @pl.when(c < n_w)
            def _():
                start_load(c, c % _P1_NBUF)

        def group_body(g, carry):
            for b in range(_P1_NBUF):
                c = g * _P1_NBUF + b
                ba = (b + _P1_AHEAD) % _P1_NBUF

                @pl.when(jnp.logical_and(c >= _P1_NBUF - _P1_AHEAD,
                                         c + _P1_AHEAD - _P1_NBUF < n_w))
                def _():
                    wait_wb(ba)

                @pl.when(c + _P1_AHEAD < n_w)
                def _():
                    start_load(c + _P1_AHEAD, ba)

                @pl.when(c < n_w)
                def _():
                    wait_load(c, b)

                    def row_body(r, rcarry):
                        for u in range(_P1_UNROLL):
                            _ln_row(bufs[b], pbufs[b],
                                    r * _P1_UNROLL + u, gj, bj)
                        return rcarry

                    lax.fori_loop(0, _P1_CHUNK // _P1_UNROLL, row_body, 0)
                    start_wb(c, b)
            return carry

        lax.fori_loop(0, n_groups, group_body, 0)

    return normalize_table


_P2_CHUNK = 128
_P2_NBUF = 3
_P2_AHEAD = 1
_P2_UNROLL = 4


def _make_gather(n_rows: int, vocab: int):
    assert n_rows % (_NW * _P2_CHUNK) == 0
    rows_per_w = n_rows // _NW
    n_chunks = rows_per_w // _P2_CHUNK
    n_full_groups = n_chunks // _P2_NBUF

    mesh = plsc.VectorSubcoreMesh(core_axis_name="c", subcore_axis_name="s")

    @functools.partial(
        pl.kernel,
        mesh=mesh,
        compiler_params=pltpu.CompilerParams(use_tc_tiling_on_sc=False),
        out_type=jax.ShapeDtypeStruct((n_rows, HIDDEN), jnp.float32),
        scratch_types=[
            pltpu.VMEM((n_chunks, _P2_CHUNK), jnp.int32),
        ] + [
            pltpu.VMEM((_P2_CHUNK, HIDDEN // 2), jnp.int32)
            for _ in range(_P2_NBUF)
        ] + [
            pltpu.VMEM((_P2_CHUNK, HIDDEN), jnp.float32)
            for _ in range(_P2_NBUF)
        ] + [pltpu.SemaphoreType.DMA for _ in range(2 * _P2_NBUF)],
    )
    def gather_rows(ids_hbm, norm_hbm, out_hbm, idx_all, *bufs_and_sems):
        gbufs = list(bufs_and_sems[:_P2_NBUF])
        obufs = list(bufs_and_sems[_P2_NBUF:2 * _P2_NBUF])
        sems = bufs_and_sems[2 * _P2_NBUF:]
        sem_g = sems[:_P2_NBUF]
        sem_w = sems[_P2_NBUF:]
        wid = lax.axis_index("s") * _NC + lax.axis_index("c")
        row_base = wid * rows_per_w
        pltpu.sync_copy(ids_hbm.at[pl.ds(wid * n_chunks, n_chunks)], idx_all)

        def start_gather(c, b):
            pltpu.make_async_copy(
                norm_hbm.at[idx_all.at[c]], gbufs[b], sem_g[b]).start()

        def wait_gather(c, b):
            pltpu.make_async_copy(
                norm_hbm.at[idx_all.at[c]], gbufs[b], sem_g[b]).wait()

        def start_wb(c, b):
            pltpu.make_async_copy(
                obufs[b],
                out_hbm.at[pl.ds(row_base + c * _P2_CHUNK, _P2_CHUNK)],
                sem_w[b]).start()

        def wait_wb(b):
            pltpu.make_async_copy(
                obufs[b], out_hbm.at[pl.ds(row_base, _P2_CHUNK)],
                sem_w[b]).wait()

        def unpack_row(gv, ov, r):
            for m in range(4):
                w = gv[r, pl.ds(16 * m, 16)]
                a = lax.shift_left(w, 16)
                b2 = w & jnp.int32(-65536)
                ov[r, pl.ds(32 * m, 16)] = \
                    lax.bitcast_convert_type(a, jnp.float32)
                ov[r, pl.ds(32 * m + 16, 16)] = \
                    lax.bitcast_convert_type(b2, jnp.float32)

        for c in range(_P2_AHEAD):
            start_gather(c, c % _P2_NBUF)

        def compute_chunk(b):
            def row_body(r, rcarry):
                for u in range(_P2_UNROLL):
                    unpack_row(gbufs[b], obufs[b], r * _P2_UNROLL + u)
                return rcarry

            lax.fori_loop(0, _P2_CHUNK // _P2_UNROLL, row_body, 0)

        def step(c, b, dynamic):
            ba = (b + _P2_AHEAD) % _P2_NBUF
            if dynamic:

                @pl.when(c >= _P2_NBUF - _P2_AHEAD)
                def _():
                    wait_wb(ba)
            elif c >= _P2_NBUF - _P2_AHEAD:
                wait_wb(ba)
            if dynamic or c + _P2_AHEAD < n_chunks:
                start_gather(c + _P2_AHEAD, ba)
            wait_gather(c, b)
            compute_chunk(b)
            start_wb(c, b)

        def group_body(g, carry):
            for b in range(_P2_NBUF):
                step(g * _P2_NBUF + b, b, True)
            return carry

        lax.fori_loop(0, n_full_groups, group_body, 0)
        for c in range(n_full_groups * _P2_NBUF, n_chunks):
            step(c, c % _P2_NBUF, False)
        for c in range(n_chunks - (_P2_NBUF - _P2_AHEAD), n_chunks):
            wait_wb(c % _P2_NBUF)

    return gather_rows


def kernel(input_ids, word_table, ln_gamma, ln_beta):
    b, l = input_ids.shape
    vocab, hidden = word_table.shape
    assert hidden == HIDDEN
    n_rows = b * l
    ids2d = input_ids.reshape(n_rows // _P2_CHUNK, _P2_CHUNK).astype(jnp.int32)
    norm_table = _make_normalize_table(vocab)(
        word_table, ln_gamma.astype(jnp.float32), ln_beta.astype(jnp.float32))
    out = _make_gather(n_rows, vocab)(ids2d, norm_table)
    return out.reshape(b, l, HIDDEN)


# speedup vs baseline: 1.6829x; 1.6829x over previous
import functools

import jax
import jax.numpy as jnp
from jax import lax
from jax.experimental import pallas as pl
from jax.experimental.pallas import tpu as pltpu
from jax.experimental.pallas import tpu_sc as plsc

HIDDEN = 128
LN_EPS = 1e-12

_info = plsc.get_sparse_core_info()
_NC, _NS, _L = _info.num_cores, _info.num_subcores, _info.num_lanes
_NW = _NC * _NS

_GATHER_DNUMS = lax.GatherDimensionNumbers(
    offset_dims=(), collapsed_slice_dims=(0,), start_index_map=(0,))


def _lane_shuffle(v, idx):
    return lax.gather(v, idx.reshape(16, 1), _GATHER_DNUMS, (1,),
                      mode=lax.GatherScatterMode.PROMISE_IN_BOUNDS)


def _xlane_sum(v):
    for sh in (8, 4, 2, 1):
        idx = lax.iota(jnp.int32, 16) ^ sh
        v = v + _lane_shuffle(v, idx)
    return v


def _ln_row(rv, r, gj, bj):
    x = [rv[r, pl.ds(16 * j, 16)] for j in range(8)]
    s1v = ((x[0] + x[1]) + (x[2] + x[3])) + ((x[4] + x[5]) + (x[6] + x[7]))
    s2v = ((x[0] * x[0] + x[1] * x[1]) + (x[2] * x[2] + x[3] * x[3])) + \
          ((x[4] * x[4] + x[5] * x[5]) + (x[6] * x[6] + x[7] * x[7]))
    s1 = _xlane_sum(s1v)
    s2 = _xlane_sum(s2v)
    mv = s1 * (1.0 / HIDDEN)
    vv = jnp.maximum(s2 * (1.0 / HIDDEN) - mv * mv, 0.0) + LN_EPS
    bi = lax.bitcast_convert_type(vv, jnp.int32)
    bi = jnp.int32(0x5F3759DF) - (bi >> 1)
    y = lax.bitcast_convert_type(bi, jnp.float32)
    hv = 0.5 * vv
    y = y * (1.5 - hv * (y * y))
    y = y * (1.5 - hv * (y * y))
    for j in range(8):
        t = y * gj[j]
        rv[r, pl.ds(16 * j, 16)] = (x[j] - mv) * t + bj[j]


_P1_CHUNK = 200
_P1_NBUF = 3
_P1_AHEAD = 1
_P1_UNROLL = 4


def _make_normalize_table(vocab: int):
    assert vocab % _P1_CHUNK == 0 and _P1_CHUNK % 8 == 0
    n_chunks = vocab // _P1_CHUNK
    max_per_w = -(-n_chunks // _NW)
    n_groups = -(-max_per_w // _P1_NBUF)

    mesh = plsc.VectorSubcoreMesh(core_axis_name="c", subcore_axis_name="s")

    @functools.partial(
        pl.kernel,
        mesh=mesh,
        out_type=jax.ShapeDtypeStruct((vocab, HIDDEN), jnp.float32),
        scratch_types=[
            pltpu.VMEM((_P1_CHUNK, HIDDEN), jnp.float32)
            for _ in range(_P1_NBUF)
        ] + [
            pltpu.VMEM((HIDDEN,), jnp.float32),
            pltpu.VMEM((HIDDEN,), jnp.float32),
        ] + [pltpu.SemaphoreType.DMA for _ in range(2 * _P1_NBUF)],
    )
    def normalize_table(table_hbm, gamma_hbm, beta_hbm, norm_hbm,
                        buf0, buf1, buf2, gamma_v, beta_v, *sems):
        bufs = [buf0, buf1, buf2]
        sem_g = sems[:_P1_NBUF]
        sem_w = sems[_P1_NBUF:]
        wid = lax.axis_index("s") * _NC + lax.axis_index("c")
        n_w = (n_chunks - wid + _NW - 1) // _NW
        pltpu.sync_copy(gamma_hbm, gamma_v)
        pltpu.sync_copy(beta_hbm, beta_v)
        gj = [gamma_v[pl.ds(16 * j, 16)] for j in range(8)]
        bj = [beta_v[pl.ds(16 * j, 16)] for j in range(8)]

        def row_off(c):
            return (wid + c * _NW) * _P1_CHUNK

        def start_load(c, b):
            pltpu.make_async_copy(
                table_hbm.at[pl.ds(row_off(c), _P1_CHUNK)],
                bufs[b], sem_g[b]).start()

        def wait_load(c, b):
            pltpu.make_async_copy(
                table_hbm.at[pl.ds(row_off(c), _P1_CHUNK)],
                bufs[b], sem_g[b]).wait()

        def start_wb(c, b):
            pltpu.make_async_copy(
                bufs[b], norm_hbm.at[pl.ds(row_off(c), _P1_CHUNK)],
                sem_w[b]).start()

        def wait_wb(b):
            pltpu.make_async_copy(
                bufs[b], norm_hbm.at[pl.ds(wid * _P1_CHUNK, _P1_CHUNK)],
                sem_w[b]).wait()

        for c in range(_P1_AHEAD):

            @pl.when(c < n_w)
            def _():
                start_load(c, c % _P1_NBUF)

        def group_body(g, carry):
            for b in range(_P1_NBUF):
                c = g * _P1_NBUF + b
                ba = (b + _P1_AHEAD) % _P1_NBUF

                @pl.when(jnp.logical_and(c >= _P1_NBUF - _P1_AHEAD,
                                         c + _P1_AHEAD - _P1_NBUF < n_w))
                def _():
                    wait_wb(ba)

                @pl.when(c + _P1_AHEAD < n_w)
                def _():
                    start_load(c + _P1_AHEAD, ba)

                @pl.when(c < n_w)
                def _():
                    wait_load(c, b)

                    def row_body(r, rcarry):
                        for u in range(_P1_UNROLL):
                            _ln_row(bufs[b], r * _P1_UNROLL + u, gj, bj)
                        return rcarry

                    lax.fori_loop(0, _P1_CHUNK // _P1_UNROLL, row_body, 0)
                    start_wb(c, b)
            return carry

        lax.fori_loop(0, n_groups, group_body, 0)

    return normalize_table


_P2_CHUNK = 128
_P2_NBUF = 5
_P2_AHEAD = 3


def _make_gather(n_rows: int, vocab: int):
    assert n_rows % (_NW * _P2_CHUNK * _P2_NBUF) == 0
    rows_per_w = n_rows // _NW
    n_chunks = rows_per_w // _P2_CHUNK

    mesh = plsc.VectorSubcoreMesh(core_axis_name="c", subcore_axis_name="s")

    @functools.partial(
        pl.kernel,
        mesh=mesh,
        out_type=jax.ShapeDtypeStruct((n_rows, HIDDEN), jnp.float32),
        scratch_types=[
            pltpu.VMEM((n_chunks, _P2_CHUNK), jnp.int32),
        ] + [
            pltpu.VMEM((_P2_CHUNK, HIDDEN), jnp.float32)
            for _ in range(_P2_NBUF)
        ] + [pltpu.SemaphoreType.DMA for _ in range(2 * _P2_NBUF)],
    )
    def gather_rows(ids_hbm, norm_hbm, out_hbm, idx_all, *bufs_and_sems):
        rows = list(bufs_and_sems[:_P2_NBUF])
        sems = bufs_and_sems[_P2_NBUF:]
        sem_g = sems[:_P2_NBUF]
        sem_w = sems[_P2_NBUF:]
        wid = lax.axis_index("s") * _NC + lax.axis_index("c")
        row_base = wid * rows_per_w
        pltpu.sync_copy(ids_hbm.at[pl.ds(wid * n_chunks, n_chunks)], idx_all)

        def start_gather(c, b):
            pltpu.make_async_copy(
                norm_hbm.at[idx_all.at[c]], rows[b], sem_g[b]).start()

        def wait_gather(c, b):
            pltpu.make_async_copy(
                norm_hbm.at[idx_all.at[c]], rows[b], sem_g[b]).wait()

        def start_wb(c, b):
            pltpu.make_async_copy(
                rows[b],
                out_hbm.at[pl.ds(row_base + c * _P2_CHUNK, _P2_CHUNK)],
                sem_w[b]).start()

        def wait_wb(b):
            pltpu.make_async_copy(
                rows[b], out_hbm.at[pl.ds(row_base, _P2_CHUNK)],
                sem_w[b]).wait()

        for c in range(_P2_AHEAD):
            start_gather(c, c % _P2_NBUF)

        def group_body(g, carry):
            for b in range(_P2_NBUF):
                c = g * _P2_NBUF + b
                ba = (b + _P2_AHEAD) % _P2_NBUF

                @pl.when(c >= _P2_NBUF - _P2_AHEAD)
                def _():
                    wait_wb(ba)

                @pl.when(c + _P2_AHEAD < n_chunks)
                def _():
                    start_gather(c + _P2_AHEAD, ba)

                wait_gather(c, b)
                start_wb(c, b)
            return carry

        lax.fori_loop(0, n_chunks // _P2_NBUF, group_body, 0)
        for c in range(n_chunks - (_P2_NBUF - _P2_AHEAD), n_chunks):
            wait_wb(c % _P2_NBUF)

    return gather_rows


def kernel(input_ids, word_table, ln_gamma, ln_beta):
    b, l = input_ids.shape
    vocab, hidden = word_table.shape
    assert hidden == HIDDEN
    n_rows = b * l
    ids2d = input_ids.reshape(n_rows // _P2_CHUNK, _P2_CHUNK).astype(jnp.int32)
    norm_table = _make_normalize_table(vocab)(
        word_table, ln_gamma.astype(jnp.float32), ln_beta.astype(jnp.float32))
    out = _make_gather(n_rows, vocab)(ids2d, norm_table)
    return out.reshape(b, l, HIDDEN)
